# baseline (speedup 1.0000x reference)
import jax
import jax.numpy as jnp
from jax import lax
from jax.experimental import pallas as pl
from jax.experimental.pallas import tpu as pltpu
from jax.experimental.pallas import tpu_sc as plsc

BATCH = 4096
SEQ = 200
EMBED = 64
VOCAB = 1000000

NUM_CORES = 2
NUM_SUBCORES = 16
NW = NUM_CORES * NUM_SUBCORES
BBLK = BATCH // NW

VCHUNKS_FULL = VOCAB // 128
VREM = VOCAB - VCHUNKS_FULL * 128
K1_ITERS = (VCHUNKS_FULL + NW - 1) // NW


def _iota16():
    return lax.iota(jnp.int32, 16)


def _k1_transpose_block(inbuf, outbuf, rows_out):
    def row_body(r, carry):
        for k in range(8):
            row_idx = _iota16() + 16 * (k % 4)
            l0 = 2 * r + (1 if k >= 4 else 0)
            col_idx = jnp.full((16,), 0, jnp.int32) + l0
            vals = plsc.load_gather(inbuf, [row_idx, col_idx])
            outbuf[r, pl.ds(16 * k, 16)] = vals
        return carry

    lax.fori_loop(0, rows_out, row_body, 0, unroll=False)


def _k1_body(table_t, out_hbm, inbuf, outbuf, inbuf2):
    wid = lax.axis_index("s") * NUM_CORES + lax.axis_index("c")

    def iter_body(ci, carry):
        c = wid + ci * NW

        @pl.when(c < VCHUNKS_FULL)
        def _():
            pltpu.sync_copy(table_t.at[:, pl.ds(c * 128, 128)], inbuf)
            _k1_transpose_block(inbuf, outbuf, 64)
            pltpu.sync_copy(outbuf, out_hbm.at[pl.ds(c * 64, 64)])

        return carry

    lax.fori_loop(0, K1_ITERS, iter_body, 0, unroll=False)

    @pl.when(wid == NW - 1)
    def _():
        pltpu.sync_copy(table_t.at[:, pl.ds(VCHUNKS_FULL * 128, VREM)], inbuf2)
        _k1_transpose_block(inbuf2, outbuf, VREM // 2)
        pltpu.sync_copy(
            outbuf.at[pl.ds(0, VREM // 2)],
            out_hbm.at[pl.ds(VCHUNKS_FULL * 64, VREM // 2)],
        )


def _k2_body(seq_hbm, table_hbm, out_hbm, idx_raw, idx_t, gbufs, tbufs,
             gsems, wsems):
    wid = lax.axis_index("s") * NUM_CORES + lax.axis_index("c")
    b0 = wid * BBLK

    pltpu.sync_copy(seq_hbm.at[pl.ds(b0, BBLK)], idx_raw)

    def idx_tr_body(s, carry):
        col = jnp.full((16,), 0, jnp.int32) + s
        for k in range(BBLK // 16):
            row_idx = _iota16() + 16 * k
            vals = plsc.load_gather(idx_raw, [row_idx, col])
            idx_t[s, pl.ds(16 * k, 16)] = vals
        return carry

    lax.fori_loop(0, SEQ, idx_tr_body, 0, unroll=False)

    def fire_gather(s, par):
        pltpu.async_copy(table_hbm.at[idx_t.at[s]], gbufs.at[par], gsems[par])

    def chunk_transpose(par):
        def e_body(e, carry):
            col = jnp.full((16,), 0, jnp.int32) + e
            for k in range(8):
                row_idx = _iota16() + 16 * k
                vals = plsc.load_gather(gbufs.at[par], [row_idx, col])
                tbufs[par, e, pl.ds(16 * k, 16)] = vals
            return carry

        lax.fori_loop(0, EMBED, e_body, 0, unroll=False)

    def fire_writeback(s, par):
        pltpu.async_copy(
            tbufs.at[par],
            out_hbm.at[s, :, pl.ds(b0, BBLK)],
            wsems[par],
        )

    def wait(sem, ref):
        pltpu.make_async_copy(table_hbm.at[pl.ds(0, 2)], ref, sem).wait()

    fire_gather(0, 0)
    fire_gather(1, 1)

    def s2_body(s2, carry):
        for par in range(2):
            s = 2 * s2 + par
            wait(gsems[par], gbufs.at[0])

            @pl.when(s2 >= 1)
            def _():
                wait(wsems[par], tbufs.at[0])

            chunk_transpose(par)
            fire_writeback(s, par)

            @pl.when(s2 < SEQ // 2 - 1)
            def _():
                fire_gather(s + 2, par)

        return carry

    lax.fori_loop(0, SEQ // 2, s2_body, 0, unroll=False)
    wait(wsems[0], tbufs.at[0])
    wait(wsems[1], tbufs.at[0])


@jax.jit
def kernel(sequence, table):
    mesh = plsc.VectorSubcoreMesh(core_axis_name="c", subcore_axis_name="s")

    transpose_run = pl.kernel(
        _k1_body,
        out_type=jax.ShapeDtypeStruct((VOCAB * EMBED // 128, 128), jnp.float32),
        mesh=mesh,
        scratch_types=[
            pltpu.VMEM((EMBED, 128), jnp.float32),
            pltpu.VMEM((EMBED, 128), jnp.float32),
            pltpu.VMEM((EMBED, VREM), jnp.float32),
        ],
        compiler_params=pltpu.CompilerParams(
            use_tc_tiling_on_sc=True, needs_layout_passes=False
        ),
    )
    table_lin = transpose_run(table.T)

    gather_run = pl.kernel(
        _k2_body,
        out_type=jax.ShapeDtypeStruct((SEQ, EMBED, BATCH), jnp.float32),
        mesh=mesh,
        scratch_types=[
            pltpu.VMEM((BBLK, SEQ), jnp.int32),
            pltpu.VMEM((SEQ, BBLK), jnp.int32),
            pltpu.VMEM((2, BBLK, EMBED), jnp.float32),
            pltpu.VMEM((2, EMBED, BBLK), jnp.float32),
            [pltpu.SemaphoreType.DMA, pltpu.SemaphoreType.DMA],
            [pltpu.SemaphoreType.DMA, pltpu.SemaphoreType.DMA],
        ],
        compiler_params=pltpu.CompilerParams(
            use_tc_tiling_on_sc=False, needs_layout_passes=False
        ),
    )
    out_phys = gather_run(
        sequence.astype(jnp.int32), table_lin.reshape(VOCAB, EMBED)
    )
    return jnp.transpose(out_phys, (2, 0, 1))


# speedup vs baseline: 1.3246x; 1.3246x over previous
import functools

import jax
import jax.numpy as jnp
from jax import lax
from jax.experimental import pallas as pl
from jax.experimental.layout import Format, Layout
from jax.experimental.pallas import tpu as pltpu
from jax.experimental.pallas import tpu_sc as plsc

BATCH = 4096
SEQ = 200
EMBED = 64
TOTAL = BATCH * SEQ
VOCAB = 1000000

NUM_CORES = 2
NUM_SUBCORES = 16
NW = NUM_CORES * NUM_SUBCORES
ROWS_PER_W = TOTAL // NW

CHUNK = 128
NCHUNK = ROWS_PER_W // CHUNK
NBUF = 4
ROUNDS = NCHUNK // NBUF

VCHUNKS_FULL = VOCAB // 128
VREM = VOCAB - VCHUNKS_FULL * 128
K1_ITERS = (VCHUNKS_FULL + NW - 1) // NW
PAD = 129
PAD2 = VREM + 1


def _iota16():
    return lax.iota(jnp.int32, 16)


def _k1_transpose_block(inbuf, outbuf, rows_out):
    def row_body(r, carry):
        for k in range(8):
            row_idx = _iota16() + 16 * (k % 4)
            l0 = 2 * r + (1 if k >= 4 else 0)
            col_idx = jnp.full((16,), 0, jnp.int32) + l0
            vals = plsc.load_gather(inbuf, [row_idx, col_idx])
            outbuf[r, pl.ds(16 * k, 16)] = vals
        return carry

    lax.fori_loop(0, rows_out, row_body, 0, unroll=False)


def _k1_body(table_t, tail_t, out_hbm, inbuf, outbuf):
    wid = lax.axis_index("s") * NUM_CORES + lax.axis_index("c")

    def iter_body(ci, carry):
        c = wid + ci * NW

        @pl.when(c < VCHUNKS_FULL)
        def _():
            pltpu.sync_copy(
                table_t.at[:, pl.ds(c * 128, 128)],
                inbuf.at[:, pl.ds(0, 128)],
            )
            _k1_transpose_block(inbuf, outbuf, 64)
            pltpu.sync_copy(outbuf, out_hbm.at[pl.ds(c * 64, 64)])

        return carry

    lax.fori_loop(0, K1_ITERS, iter_body, 0, unroll=False)

    @pl.when(wid == NW - 1)
    def _():
        pltpu.sync_copy(tail_t, inbuf.at[:, pl.ds(0, 128)])
        _k1_transpose_block(inbuf, outbuf, 64)
        pltpu.sync_copy(
            outbuf.at[pl.ds(64 - VREM // 2, VREM // 2)],
            out_hbm.at[pl.ds(VCHUNKS_FULL * 64, VREM // 2)],
        )


def _k2_body(seq_hbm, table_hbm, out_hbm, idx_v, bufs, gsem, wsem):
    wid = lax.axis_index("s") * NUM_CORES + lax.axis_index("c")
    base_chunk = wid * NCHUNK

    pltpu.sync_copy(seq_hbm.at[pl.ds(base_chunk, NCHUNK)], idx_v)

    def fire_gathers(r):
        par = lax.rem(r, 2)
        for b in range(NBUF):
            c = r * NBUF + b
            pltpu.async_copy(table_hbm.at[idx_v.at[c]], bufs.at[par, b], gsem)

    def fire_writebacks(r):
        par = lax.rem(r, 2)
        for b in range(NBUF):
            c = r * NBUF + b
            pltpu.async_copy(
                bufs.at[par, b],
                out_hbm.at[pl.ds((base_chunk + c) * CHUNK, CHUNK)],
                wsem,
            )

    def drain(sem, n):
        for _ in range(n):
            pltpu.make_async_copy(
                table_hbm.at[pl.ds(0, CHUNK)], bufs.at[0, 0], sem
            ).wait()

    fire_gathers(0)

    def round_body(r, carry):
        drain(gsem, NBUF)

        @pl.when(r >= 1)
        def _():
            drain(wsem, NBUF)

        @pl.when(r + 1 < ROUNDS)
        def _():
            fire_gathers(r + 1)

        fire_writebacks(r)
        return carry

    lax.fori_loop(0, ROUNDS, round_body, 0, unroll=False)
    drain(wsem, NBUF)


def _kernel_impl(sequence, table):
    mesh = plsc.VectorSubcoreMesh(core_axis_name="c", subcore_axis_name="s")

    transpose_run = pl.kernel(
        _k1_body,
        out_type=jax.ShapeDtypeStruct((VOCAB * EMBED // 128, 128), jnp.float32),
        mesh=mesh,
        scratch_types=[
            pltpu.VMEM((EMBED, PAD), jnp.float32),
            pltpu.VMEM((EMBED, 128), jnp.float32),
        ],
        compiler_params=pltpu.CompilerParams(
            use_tc_tiling_on_sc=True, needs_layout_passes=False
        ),
    )
    tail_t = lax.slice(table, (VOCAB - 128, 0), (VOCAB, EMBED)).T
    table_lin = transpose_run(table.T, tail_t)

    seq2d = sequence.reshape(NW * NCHUNK, CHUNK).astype(jnp.int32)
    gather_run = pl.kernel(
        _k2_body,
        out_type=jax.ShapeDtypeStruct((TOTAL, EMBED), jnp.float32),
        mesh=mesh,
        scratch_types=[
            pltpu.VMEM((NCHUNK, CHUNK), jnp.int32),
            pltpu.VMEM((2, NBUF, CHUNK, EMBED), jnp.float32),
            pltpu.SemaphoreType.DMA,
            pltpu.SemaphoreType.DMA,
        ],
        compiler_params=pltpu.CompilerParams(use_tc_tiling_on_sc=False),
    )
    out = gather_run(seq2d, table_lin.reshape(VOCAB, EMBED))
    return out.reshape(BATCH, SEQ, EMBED)


@functools.cache
def _jitted(sharding):
    if sharding is None:
        return jax.jit(_kernel_impl)
    fmt = Format(Layout(major_to_minor=(0, 1, 2), tiling=((8,),)), sharding)
    return jax.jit(_kernel_impl, out_shardings=fmt)


def kernel(sequence, table):
    try:
        sharding = table.sharding
        hash(sharding)
    except Exception:
        sharding = None
    return _jitted(sharding)(sequence, table)


# speedup vs baseline: 2.7783x; 2.0974x over previous
import functools

import jax
import jax.numpy as jnp
from jax import lax
from jax.experimental import pallas as pl
from jax.experimental.layout import Format, Layout
from jax.experimental.pallas import tpu as pltpu
from jax.experimental.pallas import tpu_sc as plsc

BATCH = 4096
SEQ = 200
EMBED = 64
TOTAL = BATCH * SEQ
VOCAB = 1000000

NUM_CORES = 2
NUM_SUBCORES = 16
NW = NUM_CORES * NUM_SUBCORES
ROWS_PER_W = TOTAL // NW

CHUNK = 128
NCHUNK = ROWS_PER_W // CHUNK
NBUF = 4
ROUNDS = NCHUNK // NBUF

VCHUNKS_FULL = VOCAB // 128
VREM = VOCAB - VCHUNKS_FULL * 128
K1_ITERS = (VCHUNKS_FULL + NW - 1) // NW
PAD = 129
PAD2 = VREM + 1


def _iota16():
    return lax.iota(jnp.int32, 16)


def _k1_transpose_block(inbuf, outbuf, rows_out):
    def row_body(r, carry):
        for k in range(8):
            row_idx = _iota16() + 16 * (k % 4)
            l0 = 2 * r + (1 if k >= 4 else 0)
            col_idx = jnp.full((16,), 0, jnp.int32) + l0
            vals = plsc.load_gather(inbuf, [row_idx, col_idx])
            outbuf[r, pl.ds(16 * k, 16)] = vals
        return carry

    lax.fori_loop(0, rows_out, row_body, 0, unroll=False)


def _k1_body(table_t, tail_t, out_hbm, inbuf, outbuf):
    wid = lax.axis_index("s") * NUM_CORES + lax.axis_index("c")

    def iter_body(ci, carry):
        c = wid + ci * NW

        @pl.when(c < VCHUNKS_FULL)
        def _():
            pltpu.sync_copy(
                table_t.at[:, pl.ds(c * 128, 128)],
                inbuf.at[:, pl.ds(0, 128)],
            )
            _k1_transpose_block(inbuf, outbuf, 64)
            pltpu.sync_copy(outbuf, out_hbm.at[pl.ds(c * 64, 64)])

        return carry

    lax.fori_loop(0, K1_ITERS, iter_body, 0, unroll=False)

    @pl.when(wid == NW - 1)
    def _():
        pltpu.sync_copy(tail_t, inbuf.at[:, pl.ds(0, 128)])
        _k1_transpose_block(inbuf, outbuf, 64)
        pltpu.sync_copy(
            outbuf.at[pl.ds(64 - VREM // 2, VREM // 2)],
            out_hbm.at[pl.ds(VCHUNKS_FULL * 64, VREM // 2)],
        )


def _k2_body(seq_hbm, table_hbm, out_hbm, idx_v, bufs, gsem, wsem):
    wid = lax.axis_index("s") * NUM_CORES + lax.axis_index("c")
    base_chunk = wid * NCHUNK

    pltpu.sync_copy(seq_hbm.at[pl.ds(base_chunk, NCHUNK)], idx_v)

    def fire_gathers(r):
        par = lax.rem(r, 2)
        for b in range(NBUF):
            c = r * NBUF + b
            pltpu.async_copy(table_hbm.at[idx_v.at[c]], bufs.at[par, b], gsem)

    def fire_writebacks(r):
        par = lax.rem(r, 2)
        for b in range(NBUF):
            c = r * NBUF + b
            pltpu.async_copy(
                bufs.at[par, b],
                out_hbm.at[pl.ds((base_chunk + c) * CHUNK, CHUNK)],
                wsem,
            )

    def drain(sem, n):
        for _ in range(n):
            pltpu.make_async_copy(
                table_hbm.at[pl.ds(0, CHUNK)], bufs.at[0, 0], sem
            ).wait()

    fire_gathers(0)

    def round_body(r, carry):
        drain(gsem, NBUF)

        @pl.when(r >= 1)
        def _():
            drain(wsem, NBUF)

        @pl.when(r + 1 < ROUNDS)
        def _():
            fire_gathers(r + 1)

        fire_writebacks(r)
        return carry

    lax.fori_loop(0, ROUNDS, round_body, 0, unroll=False)
    drain(wsem, NBUF)


TC_BLK = 2048
TC_GRID = (VOCAB + TC_BLK - 1) // TC_BLK


def _k1_tc_body(in_ref, out_ref):
    t = jnp.transpose(in_ref[...], (1, 0))
    t3 = t.reshape(TC_BLK // 2, 2, EMBED)
    out_ref[...] = jnp.concatenate([t3[:, 0, :], t3[:, 1, :]], axis=-1)


def _kernel_impl(sequence, table):
    mesh = plsc.VectorSubcoreMesh(core_axis_name="c", subcore_axis_name="s")

    table_lin = pl.pallas_call(
        _k1_tc_body,
        out_shape=jax.ShapeDtypeStruct((VOCAB * EMBED // 128, 128), jnp.float32),
        grid=(TC_GRID,),
        in_specs=[
            pl.BlockSpec((EMBED, TC_BLK), lambda c: (0, c)),
        ],
        out_specs=pl.BlockSpec((TC_BLK // 2, 128), lambda c: (c, 0)),
    )(table.T)

    seq2d = sequence.reshape(NW * NCHUNK, CHUNK).astype(jnp.int32)
    gather_run = pl.kernel(
        _k2_body,
        out_type=jax.ShapeDtypeStruct((TOTAL, EMBED), jnp.float32),
        mesh=mesh,
        scratch_types=[
            pltpu.VMEM((NCHUNK, CHUNK), jnp.int32),
            pltpu.VMEM((2, NBUF, CHUNK, EMBED), jnp.float32),
            pltpu.SemaphoreType.DMA,
            pltpu.SemaphoreType.DMA,
        ],
        compiler_params=pltpu.CompilerParams(use_tc_tiling_on_sc=False),
    )
    out = gather_run(seq2d, table_lin.reshape(VOCAB, EMBED))
    return out.reshape(BATCH, SEQ, EMBED)


@functools.cache
def _jitted(sharding):
    if sharding is None:
        return jax.jit(_kernel_impl)
    fmt = Format(Layout(major_to_minor=(0, 1, 2), tiling=((8,),)), sharding)
    return jax.jit(_kernel_impl, out_shardings=fmt)


def kernel(sequence, table):
    try:
        sharding = table.sharding
        hash(sharding)
    except Exception:
        sharding = None
    return _jitted(sharding)(sequence, table)


# speedup vs baseline: 3.1319x; 1.1273x over previous
import functools

import jax
import jax.numpy as jnp
from jax import lax
from jax.experimental import pallas as pl
from jax.experimental.layout import Format, Layout, with_layout_constraint
from jax.experimental.pallas import tpu as pltpu
from jax.experimental.pallas import tpu_sc as plsc

BATCH = 4096
SEQ = 200
EMBED = 64
TOTAL = BATCH * SEQ
VOCAB = 1000000

NUM_CORES = 2
NUM_SUBCORES = 16
NW = NUM_CORES * NUM_SUBCORES
ROWS_PER_W = TOTAL // NW

CHUNK = 128
NCHUNK = ROWS_PER_W // CHUNK
NBUF = 4
ROUNDS = NCHUNK // NBUF

VCHUNKS_FULL = VOCAB // 128
VREM = VOCAB - VCHUNKS_FULL * 128
K1_ITERS = (VCHUNKS_FULL + NW - 1) // NW
PAD = 129
PAD2 = VREM + 1


def _iota16():
    return lax.iota(jnp.int32, 16)


def _k1_transpose_block(inbuf, outbuf, rows_out):
    def row_body(r, carry):
        for k in range(8):
            row_idx = _iota16() + 16 * (k % 4)
            l0 = 2 * r + (1 if k >= 4 else 0)
            col_idx = jnp.full((16,), 0, jnp.int32) + l0
            vals = plsc.load_gather(inbuf, [row_idx, col_idx])
            outbuf[r, pl.ds(16 * k, 16)] = vals
        return carry

    lax.fori_loop(0, rows_out, row_body, 0, unroll=False)


def _k1_body(table_t, tail_t, out_hbm, inbuf, outbuf):
    wid = lax.axis_index("s") * NUM_CORES + lax.axis_index("c")

    def iter_body(ci, carry):
        c = wid + ci * NW

        @pl.when(c < VCHUNKS_FULL)
        def _():
            pltpu.sync_copy(
                table_t.at[:, pl.ds(c * 128, 128)],
                inbuf.at[:, pl.ds(0, 128)],
            )
            _k1_transpose_block(inbuf, outbuf, 64)
            pltpu.sync_copy(outbuf, out_hbm.at[pl.ds(c * 64, 64)])

        return carry

    lax.fori_loop(0, K1_ITERS, iter_body, 0, unroll=False)

    @pl.when(wid == NW - 1)
    def _():
        pltpu.sync_copy(tail_t, inbuf.at[:, pl.ds(0, 128)])
        _k1_transpose_block(inbuf, outbuf, 64)
        pltpu.sync_copy(
            outbuf.at[pl.ds(64 - VREM // 2, VREM // 2)],
            out_hbm.at[pl.ds(VCHUNKS_FULL * 64, VREM // 2)],
        )


def _k2_body(seq_hbm, table_hbm, out_hbm, idx_v, bufs, gsem, wsem):
    wid = lax.axis_index("s") * NUM_CORES + lax.axis_index("c")
    base_chunk = wid * NCHUNK

    pltpu.sync_copy(seq_hbm.at[pl.ds(base_chunk, NCHUNK)], idx_v)

    def fire_gathers(r):
        par = lax.rem(r, 2)
        for b in range(NBUF):
            c = r * NBUF + b
            pltpu.async_copy(table_hbm.at[idx_v.at[c]], bufs.at[par, b], gsem)

    def fire_writebacks(r):
        par = lax.rem(r, 2)
        for b in range(NBUF):
            c = r * NBUF + b
            pltpu.async_copy(
                bufs.at[par, b],
                out_hbm.at[pl.ds((base_chunk + c) * CHUNK, CHUNK)],
                wsem,
            )

    def drain(sem, n):
        for _ in range(n):
            pltpu.make_async_copy(
                table_hbm.at[pl.ds(0, CHUNK)], bufs.at[0, 0], sem
            ).wait()

    fire_gathers(0)

    def round_body(r, carry):
        drain(gsem, NBUF)

        @pl.when(r >= 1)
        def _():
            drain(wsem, NBUF)

        @pl.when(r + 1 < ROUNDS)
        def _():
            fire_gathers(r + 1)

        fire_writebacks(r)
        return carry

    lax.fori_loop(0, ROUNDS, round_body, 0, unroll=False)
    drain(wsem, NBUF)


TC_BLK = 8192
TC_GRID = (VOCAB + TC_BLK - 1) // TC_BLK


def _k1_tc_body(in_ref, out_ref):
    t = jnp.transpose(in_ref[...], (1, 0))
    t3 = t.reshape(TC_BLK // 2, 2, EMBED)
    out_ref[...] = jnp.concatenate([t3[:, 0, :], t3[:, 1, :]], axis=-1)


BBLK = BATCH // NW


def _k3_tc_body(in_ref, out_ref):
    x = in_ref[...].reshape(BBLK, SEQ // 2, 2, EMBED)
    x = jnp.transpose(x, (1, 2, 3, 0))
    out_ref[...] = x.reshape(SEQ, EMBED, BBLK)


def _kernel_impl(sequence, table):
    mesh = plsc.VectorSubcoreMesh(core_axis_name="c", subcore_axis_name="s")

    table_lin = pl.pallas_call(
        _k1_tc_body,
        out_shape=jax.ShapeDtypeStruct((VOCAB * EMBED // 128, 128), jnp.float32),
        grid=(TC_GRID,),
        in_specs=[
            pl.BlockSpec((EMBED, TC_BLK), lambda c: (0, c)),
        ],
        out_specs=pl.BlockSpec((TC_BLK // 2, 128), lambda c: (c, 0)),
    )(table.T)

    seq2d = sequence.reshape(NW * NCHUNK, CHUNK).astype(jnp.int32)
    gather_run = pl.kernel(
        _k2_body,
        out_type=jax.ShapeDtypeStruct((TOTAL, EMBED), jnp.float32),
        mesh=mesh,
        scratch_types=[
            pltpu.VMEM((NCHUNK, CHUNK), jnp.int32),
            pltpu.VMEM((2, NBUF, CHUNK, EMBED), jnp.float32),
            pltpu.SemaphoreType.DMA,
            pltpu.SemaphoreType.DMA,
        ],
        compiler_params=pltpu.CompilerParams(use_tc_tiling_on_sc=False),
    )
    out = gather_run(seq2d, table_lin.reshape(VOCAB, EMBED))
    return out.reshape(BATCH, SEQ, EMBED)


@functools.cache
def _jitted(sharding):
    if sharding is None:
        return jax.jit(_kernel_impl)
    fmt = Format(Layout(major_to_minor=(0, 1, 2), tiling=((8,),)), sharding)
    return jax.jit(_kernel_impl, out_shardings=fmt)


def kernel(sequence, table):
    try:
        sharding = table.sharding
        hash(sharding)
    except Exception:
        sharding = None
    return _jitted(sharding)(sequence, table)


# speedup vs baseline: 4.4433x; 1.4187x over previous
import functools

import jax
import jax.numpy as jnp
from jax import lax
from jax.experimental import pallas as pl
from jax.experimental.layout import Format, Layout, with_layout_constraint
from jax.experimental.pallas import tpu as pltpu
from jax.experimental.pallas import tpu_sc as plsc

BATCH = 4096
SEQ = 200
EMBED = 64
TOTAL = BATCH * SEQ
VOCAB = 1000000

NUM_CORES = 2
NUM_SUBCORES = 16
NW = NUM_CORES * NUM_SUBCORES
ROWS_PER_W = TOTAL // NW

CHUNK = 128
NCHUNK = ROWS_PER_W // CHUNK
NBUF = 4
ROUNDS = NCHUNK // NBUF

VCHUNKS_FULL = VOCAB // 128
VREM = VOCAB - VCHUNKS_FULL * 128
K1_ITERS = (VCHUNKS_FULL + NW - 1) // NW
PAD = 129
PAD2 = VREM + 1


def _iota16():
    return lax.iota(jnp.int32, 16)


def _k1_transpose_block(inbuf, outbuf, rows_out):
    def row_body(r, carry):
        for k in range(8):
            row_idx = _iota16() + 16 * (k % 4)
            l0 = 2 * r + (1 if k >= 4 else 0)
            col_idx = jnp.full((16,), 0, jnp.int32) + l0
            vals = plsc.load_gather(inbuf, [row_idx, col_idx])
            outbuf[r, pl.ds(16 * k, 16)] = vals
        return carry

    lax.fori_loop(0, rows_out, row_body, 0, unroll=False)


def _k1_body(table_t, tail_t, out_hbm, inbuf, outbuf):
    wid = lax.axis_index("s") * NUM_CORES + lax.axis_index("c")

    def iter_body(ci, carry):
        c = wid + ci * NW

        @pl.when(c < VCHUNKS_FULL)
        def _():
            pltpu.sync_copy(
                table_t.at[:, pl.ds(c * 128, 128)],
                inbuf.at[:, pl.ds(0, 128)],
            )
            _k1_transpose_block(inbuf, outbuf, 64)
            pltpu.sync_copy(outbuf, out_hbm.at[pl.ds(c * 64, 64)])

        return carry

    lax.fori_loop(0, K1_ITERS, iter_body, 0, unroll=False)

    @pl.when(wid == NW - 1)
    def _():
        pltpu.sync_copy(tail_t, inbuf.at[:, pl.ds(0, 128)])
        _k1_transpose_block(inbuf, outbuf, 64)
        pltpu.sync_copy(
            outbuf.at[pl.ds(64 - VREM // 2, VREM // 2)],
            out_hbm.at[pl.ds(VCHUNKS_FULL * 64, VREM // 2)],
        )


def _k2_body(seq_hbm, table_hbm, out_hbm, idx_v, bufs, gsem, wsem):
    wid = lax.axis_index("s") * NUM_CORES + lax.axis_index("c")
    base_chunk = wid * NCHUNK

    pltpu.sync_copy(seq_hbm.at[pl.ds(base_chunk, NCHUNK)], idx_v)

    def fire_gathers(r):
        par = lax.rem(r, 2)
        for b in range(NBUF):
            c = r * NBUF + b
            pltpu.async_copy(table_hbm.at[idx_v.at[c]], bufs.at[par, b], gsem)

    def fire_writebacks(r):
        par = lax.rem(r, 2)
        for b in range(NBUF):
            c = r * NBUF + b
            pltpu.async_copy(
                bufs.at[par, b],
                out_hbm.at[pl.ds((base_chunk + c) * CHUNK, CHUNK),
                           pl.ds(0, EMBED)],
                wsem,
            )

    def drain(sem, n):
        for _ in range(n):
            pltpu.make_async_copy(
                table_hbm.at[pl.ds(0, CHUNK)], bufs.at[0, 0], sem
            ).wait()

    fire_gathers(0)

    def round_body(r, carry):
        drain(gsem, NBUF)

        @pl.when(r >= 1)
        def _():
            drain(wsem, NBUF)

        @pl.when(r + 1 < ROUNDS)
        def _():
            fire_gathers(r + 1)

        fire_writebacks(r)
        return carry

    lax.fori_loop(0, ROUNDS, round_body, 0, unroll=False)
    drain(wsem, NBUF)


TC_BLK = 8192
TC_GRID = (VOCAB + TC_BLK - 1) // TC_BLK


def _k1_tc_body(in_ref, out_ref):
    t = jnp.transpose(in_ref[...], (1, 0))
    t3 = t.reshape(TC_BLK // 2, 2, EMBED)
    out_ref[...] = jnp.concatenate([t3[:, 0, :], t3[:, 1, :]], axis=-1)


BBLK = BATCH // NW


def _k3_tc_body(in_ref, out_ref):
    x = in_ref[...].reshape(BBLK, SEQ // 2, 2, EMBED)
    x = jnp.transpose(x, (1, 2, 3, 0))
    out_ref[...] = x.reshape(SEQ, EMBED, BBLK)


def _kernel_impl(sequence, table):
    mesh = plsc.VectorSubcoreMesh(core_axis_name="c", subcore_axis_name="s")

    table_lin = pl.pallas_call(
        _k1_tc_body,
        out_shape=jax.ShapeDtypeStruct((VOCAB * EMBED // 128, 128), jnp.float32),
        grid=(TC_GRID,),
        in_specs=[
            pl.BlockSpec((EMBED, TC_BLK), lambda c: (0, c)),
        ],
        out_specs=pl.BlockSpec((TC_BLK // 2, 128), lambda c: (c, 0)),
    )(table.T)

    seq2d = sequence.reshape(NW * NCHUNK, CHUNK).astype(jnp.int32)
    gather_run = pl.kernel(
        _k2_body,
        out_type=jax.ShapeDtypeStruct((TOTAL, 128), jnp.float32),
        mesh=mesh,
        scratch_types=[
            pltpu.VMEM((NCHUNK, CHUNK), jnp.int32),
            pltpu.VMEM((2, NBUF, CHUNK, EMBED), jnp.float32),
            pltpu.SemaphoreType.DMA,
            pltpu.SemaphoreType.DMA,
        ],
        compiler_params=pltpu.CompilerParams(use_tc_tiling_on_sc=False),
    )
    out_pad = gather_run(seq2d, table_lin.reshape(VOCAB, EMBED))
    out = lax.slice(out_pad, (0, 0), (TOTAL, EMBED))
    return out.reshape(BATCH, SEQ, EMBED)


@functools.cache
def _jitted(sharding):
    if sharding is None:
        return jax.jit(_kernel_impl)
    fmt = Format(Layout(major_to_minor=(0, 1, 2), tiling=((8,),)), sharding)
    return jax.jit(_kernel_impl, out_shardings=fmt)


def kernel(sequence, table):
    try:
        sharding = table.sharding
        hash(sharding)
    except Exception:
        sharding = None
    return _jitted(sharding)(sequence, table)


# speedup vs baseline: 4.4561x; 1.0029x over previous
import functools

import jax
import jax.numpy as jnp
from jax import lax
from jax.experimental import pallas as pl
from jax.experimental.layout import Format, Layout, with_layout_constraint
from jax.experimental.pallas import tpu as pltpu
from jax.experimental.pallas import tpu_sc as plsc

BATCH = 4096
SEQ = 200
EMBED = 64
TOTAL = BATCH * SEQ
VOCAB = 1000000

NUM_CORES = 2
NUM_SUBCORES = 16
NW = NUM_CORES * NUM_SUBCORES
ROWS_PER_W = TOTAL // NW

CHUNK = 128
NCHUNK = ROWS_PER_W // CHUNK
NBUF = 4
ROUNDS = NCHUNK // NBUF

VCHUNKS_FULL = VOCAB // 128
VREM = VOCAB - VCHUNKS_FULL * 128
K1_ITERS = (VCHUNKS_FULL + NW - 1) // NW
PAD = 129
PAD2 = VREM + 1


def _iota16():
    return lax.iota(jnp.int32, 16)


def _k1_transpose_block(inbuf, outbuf, rows_out):
    def row_body(r, carry):
        for k in range(8):
            row_idx = _iota16() + 16 * (k % 4)
            l0 = 2 * r + (1 if k >= 4 else 0)
            col_idx = jnp.full((16,), 0, jnp.int32) + l0
            vals = plsc.load_gather(inbuf, [row_idx, col_idx])
            outbuf[r, pl.ds(16 * k, 16)] = vals
        return carry

    lax.fori_loop(0, rows_out, row_body, 0, unroll=False)


def _k1_body(table_t, tail_t, out_hbm, inbuf, outbuf):
    wid = lax.axis_index("s") * NUM_CORES + lax.axis_index("c")

    def iter_body(ci, carry):
        c = wid + ci * NW

        @pl.when(c < VCHUNKS_FULL)
        def _():
            pltpu.sync_copy(
                table_t.at[:, pl.ds(c * 128, 128)],
                inbuf.at[:, pl.ds(0, 128)],
            )
            _k1_transpose_block(inbuf, outbuf, 64)
            pltpu.sync_copy(outbuf, out_hbm.at[pl.ds(c * 64, 64)])

        return carry

    lax.fori_loop(0, K1_ITERS, iter_body, 0, unroll=False)

    @pl.when(wid == NW - 1)
    def _():
        pltpu.sync_copy(tail_t, inbuf.at[:, pl.ds(0, 128)])
        _k1_transpose_block(inbuf, outbuf, 64)
        pltpu.sync_copy(
            outbuf.at[pl.ds(64 - VREM // 2, VREM // 2)],
            out_hbm.at[pl.ds(VCHUNKS_FULL * 64, VREM // 2)],
        )


def _k2_body(seq_hbm, table_hbm, out_hbm, idx_v, bufs, gsem, wsem):
    wid = lax.axis_index("s") * NUM_CORES + lax.axis_index("c")
    base_chunk = wid * NCHUNK

    pltpu.sync_copy(seq_hbm.at[pl.ds(base_chunk, NCHUNK)], idx_v)

    def fire_gathers(r):
        par = lax.rem(r, 2)
        for b in range(NBUF):
            c = r * NBUF + b
            pltpu.async_copy(table_hbm.at[idx_v.at[c]], bufs.at[par, b], gsem)

    def fire_writebacks(r):
        par = lax.rem(r, 2)
        for b in range(NBUF):
            c = r * NBUF + b
            pltpu.async_copy(
                bufs.at[par, b],
                out_hbm.at[pl.ds((base_chunk + c) * CHUNK, CHUNK),
                           pl.ds(0, EMBED)],
                wsem,
            )

    def drain(sem, n):
        for _ in range(n):
            pltpu.make_async_copy(
                table_hbm.at[pl.ds(0, CHUNK)], bufs.at[0, 0], sem
            ).wait()

    fire_gathers(0)

    def round_body(r, carry):
        drain(gsem, NBUF)

        @pl.when(r >= 1)
        def _():
            drain(wsem, NBUF)

        @pl.when(r + 1 < ROUNDS)
        def _():
            fire_gathers(r + 1)

        fire_writebacks(r)
        return carry

    lax.fori_loop(0, ROUNDS, round_body, 0, unroll=False)
    drain(wsem, NBUF)


TC_BLK = 16384
TC_GRID = (VOCAB + TC_BLK - 1) // TC_BLK


def _k1_tc_body(in_ref, out_ref):
    t = jnp.transpose(in_ref[...], (1, 0))
    t3 = t.reshape(TC_BLK // 2, 2, EMBED)
    out_ref[...] = jnp.concatenate([t3[:, 0, :], t3[:, 1, :]], axis=-1)


BBLK = BATCH // NW


def _k3_tc_body(in_ref, out_ref):
    x = in_ref[...].reshape(BBLK, SEQ // 2, 2, EMBED)
    x = jnp.transpose(x, (1, 2, 3, 0))
    out_ref[...] = x.reshape(SEQ, EMBED, BBLK)


def _kernel_impl(sequence, table):
    mesh = plsc.VectorSubcoreMesh(core_axis_name="c", subcore_axis_name="s")

    table_lin = pl.pallas_call(
        _k1_tc_body,
        out_shape=jax.ShapeDtypeStruct((VOCAB * EMBED // 128, 128), jnp.float32),
        grid=(TC_GRID,),
        in_specs=[
            pl.BlockSpec((EMBED, TC_BLK), lambda c: (0, c)),
        ],
        out_specs=pl.BlockSpec((TC_BLK // 2, 128), lambda c: (c, 0)),
    )(table.T)

    seq2d = sequence.reshape(NW * NCHUNK, CHUNK).astype(jnp.int32)
    gather_run = pl.kernel(
        _k2_body,
        out_type=jax.ShapeDtypeStruct((TOTAL, 128), jnp.float32),
        mesh=mesh,
        scratch_types=[
            pltpu.VMEM((NCHUNK, CHUNK), jnp.int32),
            pltpu.VMEM((2, NBUF, CHUNK, EMBED), jnp.float32),
            pltpu.SemaphoreType.DMA,
            pltpu.SemaphoreType.DMA,
        ],
        compiler_params=pltpu.CompilerParams(use_tc_tiling_on_sc=False),
    )
    out_pad = gather_run(seq2d, table_lin.reshape(VOCAB, EMBED))
    out = lax.slice(out_pad, (0, 0), (TOTAL, EMBED))
    return out.reshape(BATCH, SEQ, EMBED)


@functools.cache
def _jitted(sharding):
    if sharding is None:
        return jax.jit(_kernel_impl)
    fmt = Format(Layout(major_to_minor=(0, 1, 2), tiling=((8,),)), sharding)
    return jax.jit(_kernel_impl, out_shardings=fmt)


def kernel(sequence, table):
    try:
        sharding = table.sharding
        hash(sharding)
    except Exception:
        sharding = None
    return _jitted(sharding)(sequence, table)


# speedup vs baseline: 4.4579x; 1.0004x over previous
import jax
import jax.numpy as jnp
from jax import lax
from jax.experimental import pallas as pl
from jax.experimental.pallas import tpu as pltpu
from jax.experimental.pallas import tpu_sc as plsc

BATCH = 4096
SEQ = 200
EMBED = 64
TOTAL = BATCH * SEQ
VOCAB = 1000000

NUM_CORES = 2
NUM_SUBCORES = 16
NW = NUM_CORES * NUM_SUBCORES
ROWS_PER_W = TOTAL // NW

CHUNK = 128
NCHUNK = ROWS_PER_W // CHUNK
NBUF = 4
ROUNDS = NCHUNK // NBUF

TC_BLK = 16384
TC_GRID = (VOCAB + TC_BLK - 1) // TC_BLK


def _relayout_tc_body(in_ref, out_ref):
    t = jnp.transpose(in_ref[...], (1, 0))
    t3 = t.reshape(TC_BLK // 2, 2, EMBED)
    out_ref[...] = jnp.concatenate([t3[:, 0, :], t3[:, 1, :]], axis=-1)


def _gather_sc_body(seq_hbm, table_hbm, out_hbm, idx_v, bufs, gsem, wsem):
    wid = lax.axis_index("s") * NUM_CORES + lax.axis_index("c")
    base_chunk = wid * NCHUNK

    pltpu.sync_copy(seq_hbm.at[pl.ds(base_chunk, NCHUNK)], idx_v)

    def fire_gathers(r):
        par = lax.rem(r, 2)
        for b in range(NBUF):
            c = r * NBUF + b
            pltpu.async_copy(table_hbm.at[idx_v.at[c]], bufs.at[par, b], gsem)

    def fire_writebacks(r):
        par = lax.rem(r, 2)
        for b in range(NBUF):
            c = r * NBUF + b
            pltpu.async_copy(
                bufs.at[par, b],
                out_hbm.at[pl.ds((base_chunk + c) * CHUNK, CHUNK),
                           pl.ds(0, EMBED)],
                wsem,
            )

    def drain(sem, n):
        for _ in range(n):
            pltpu.make_async_copy(
                table_hbm.at[pl.ds(0, CHUNK)], bufs.at[0, 0], sem
            ).wait()

    fire_gathers(0)

    def round_body(r, carry):
        drain(gsem, NBUF)

        @pl.when(r >= 1)
        def _():
            drain(wsem, NBUF)

        @pl.when(r + 1 < ROUNDS)
        def _():
            fire_gathers(r + 1)

        fire_writebacks(r)
        return carry

    lax.fori_loop(0, ROUNDS, round_body, 0, unroll=False)
    drain(wsem, NBUF)


@jax.jit
def kernel(sequence, table):
    table_lin = pl.pallas_call(
        _relayout_tc_body,
        out_shape=jax.ShapeDtypeStruct((VOCAB * EMBED // 128, 128),
                                       jnp.float32),
        grid=(TC_GRID,),
        in_specs=[
            pl.BlockSpec((EMBED, TC_BLK), lambda c: (0, c)),
        ],
        out_specs=pl.BlockSpec((TC_BLK // 2, 128), lambda c: (c, 0)),
    )(table.T)

    seq2d = sequence.reshape(NW * NCHUNK, CHUNK).astype(jnp.int32)
    gather_run = pl.kernel(
        _gather_sc_body,
        out_type=jax.ShapeDtypeStruct((TOTAL, 128), jnp.float32),
        mesh=plsc.VectorSubcoreMesh(core_axis_name="c", subcore_axis_name="s"),
        scratch_types=[
            pltpu.VMEM((NCHUNK, CHUNK), jnp.int32),
            pltpu.VMEM((2, NBUF, CHUNK, EMBED), jnp.float32),
            pltpu.SemaphoreType.DMA,
            pltpu.SemaphoreType.DMA,
        ],
        compiler_params=pltpu.CompilerParams(use_tc_tiling_on_sc=False),
    )
    out_pad = gather_run(seq2d, table_lin.reshape(VOCAB, EMBED))
    out = lax.slice(out_pad, (0, 0), (TOTAL, EMBED))
    return out.reshape(BATCH, SEQ, EMBED)
